# baseline (speedup 1.0000x reference)
import functools
import math

import jax
import jax.numpy as jnp
from jax import lax
from jax.experimental import pallas as pl
from jax.experimental.pallas import tpu as pltpu
from jax.experimental.pallas import tpu_sc as plsc

_LANES = 16


@functools.lru_cache(maxsize=None)
def _build_sc_lookup(b: int, l: int, vocab: int, d: int):
    info = plsc.get_sparse_core_info()
    nc, ns = info.num_cores, info.num_subcores
    nw = nc * ns
    bw = b // nw
    assert bw * nw == b and bw == 128
    assert d == 64 and l % 2 == 0
    scale = math.sqrt(d)
    n_pairs = l // 2

    mesh = plsc.VectorSubcoreMesh(core_axis_name="c", subcore_axis_name="s")

    @functools.partial(
        pl.kernel,
        mesh=mesh,
        out_type=jax.ShapeDtypeStruct((l, d // 8, nw, 8, 128), jnp.float32),
        scratch_types=[
            pltpu.VMEM((l, bw), jnp.int32),
            pltpu.VMEM((l, bw), jnp.int32),
            pltpu.VMEM((bw, d), jnp.float32),
            pltpu.VMEM((bw, d), jnp.float32),
            pltpu.VMEM((d, bw), jnp.float32),
            pltpu.VMEM((d, bw), jnp.float32),
            pltpu.SemaphoreType.DMA,
            pltpu.SemaphoreType.DMA,
            pltpu.SemaphoreType.DMA,
            pltpu.SemaphoreType.DMA,
        ],
        compiler_params=pltpu.CompilerParams(
            needs_layout_passes=False, use_tc_tiling_on_sc=False),
    )
    def k(tok_hbm, lutr_hbm, out_hbm, idx_v, idx2_v, rows_0, rows_1,
          blk_a, blk_b, gsem_0, gsem_1, osem_a, osem_b):
        wid = lax.axis_index("s") * nc + lax.axis_index("c")
        pltpu.sync_copy(tok_hbm.at[:, pl.ds(wid * bw, bw)], idx_v)

        lanes = jax.lax.iota(jnp.int32, 16)

        def shift_body(i, carry):
            for g in range(bw // _LANES):
                sl = (i, pl.ds(g * _LANES, _LANES))
                idx2_v[sl] = lax.shift_left(idx_v[sl], 1)
            return carry

        lax.fori_loop(0, l, shift_body, 0)

        def fire_gather(li, rows, gsem):
            return pltpu.async_copy(lutr_hbm.at[idx2_v.at[li]], rows, gsem)

        def wait_gather(rows, gsem):
            pltpu.make_async_copy(lutr_hbm.at[pl.ds(0, bw)], rows, gsem).wait()

        def wait_scatter(blk, osem):
            for dt in range(d // 8):
                pltpu.make_async_copy(
                    out_hbm.at[0, 0, 0], blk.at[pl.ds(dt * 8, 8), :], osem
                ).wait()

        n_grp = bw // _LANES

        def extract(li, rows, blk):
            rowvs = [lanes + (g * _LANES) for g in range(n_grp)]

            def obody(o, carry):
                rvs = carry
                rot = jnp.bitwise_and(lanes + o, 15)
                for c in range(d // _LANES):
                    ddv = rot + (c * _LANES)
                    for g in range(n_grp):
                        val = plsc.load_gather(rows, [rvs[g], ddv])
                        plsc.store_scatter(blk, [ddv, rvs[g]], val * scale)
                return rvs

            lax.fori_loop(0, _LANES, obody, tuple(rowvs))

        def fire_scatter(li, blk, osem):
            for dt in range(d // 8):
                pltpu.async_copy(
                    blk.at[pl.ds(dt * 8, 8), :], out_hbm.at[li, dt, wid], osem
                )

        fire_gather(0, rows_0, gsem_0)
        fire_gather(1, rows_1, gsem_1)

        def pair_body(t, carry):
            l0 = t * 2
            wait_gather(rows_0, gsem_0)

            @pl.when(t > 0)
            def _():
                wait_scatter(blk_a, osem_a)

            extract(l0, rows_0, blk_a)
            fire_scatter(l0, blk_a, osem_a)

            @pl.when(t < n_pairs - 1)
            def _():
                fire_gather(l0 + 2, rows_0, gsem_0)

            wait_gather(rows_1, gsem_1)

            @pl.when(t > 0)
            def _():
                wait_scatter(blk_b, osem_b)

            extract(l0 + 1, rows_1, blk_b)
            fire_scatter(l0 + 1, blk_b, osem_b)

            @pl.when(t < n_pairs - 1)
            def _():
                fire_gather(l0 + 3, rows_1, gsem_1)

            return carry

        lax.fori_loop(0, n_pairs, pair_body, 0)
        wait_scatter(blk_a, osem_a)
        wait_scatter(blk_b, osem_b)

    return k


def kernel(input_tokens, lut):
    b, l = input_tokens.shape
    vocab, d = lut.shape
    tok_t = input_tokens.astype(jnp.int32).T
    lutp = jnp.pad(lut, ((0, 0), (0, d))).reshape(2 * vocab, d)
    out5 = _build_sc_lookup(b, l, vocab, d)(tok_t, lutp)
    return out5.transpose(2, 4, 0, 1, 3).reshape(b, l, d)


# speedup vs baseline: 1.1617x; 1.1617x over previous
import functools
import math

import jax
import jax.numpy as jnp
from jax import lax
from jax.experimental import pallas as pl
from jax.experimental.pallas import tpu as pltpu
from jax.experimental.pallas import tpu_sc as plsc

_LANES = 16
_BV = 2048


@functools.lru_cache(maxsize=None)
def _build_tc_relayout(vocab: int, d: int):
    scale = math.sqrt(d)

    def body(in_ref, out_ref):
        rows = in_ref[...].T * scale
        pad = jnp.zeros((_BV, 2 * d - d), jnp.float32)
        out_ref[...] = jnp.concatenate([rows, pad], axis=1)

    return pl.pallas_call(
        body,
        grid=(pl.cdiv(vocab, _BV),),
        in_specs=[pl.BlockSpec((d, _BV), lambda i: (0, i))],
        out_specs=pl.BlockSpec((_BV, 2 * d), lambda i: (i, 0)),
        out_shape=jax.ShapeDtypeStruct((vocab, 2 * d), jnp.float32),
    )


@functools.lru_cache(maxsize=None)
def _build_sc_lookup(b: int, l: int, vocab: int, d: int):
    info = plsc.get_sparse_core_info()
    nc, ns = info.num_cores, info.num_subcores
    nw = nc * ns
    bw = b // nw
    assert bw * nw == b and bw == 128
    assert d == 64 and l % 2 == 0
    scale = math.sqrt(d)
    n_pairs = l // 2

    mesh = plsc.VectorSubcoreMesh(core_axis_name="c", subcore_axis_name="s")

    @functools.partial(
        pl.kernel,
        mesh=mesh,
        out_type=jax.ShapeDtypeStruct((l, d // 8, nw, 8, 128), jnp.float32),
        scratch_types=[
            pltpu.VMEM((l, bw), jnp.int32),
            pltpu.VMEM((l, bw), jnp.int32),
            pltpu.VMEM((bw, d), jnp.float32),
            pltpu.VMEM((bw, d), jnp.float32),
            pltpu.VMEM((d, bw), jnp.float32),
            pltpu.VMEM((d, bw), jnp.float32),
            pltpu.SemaphoreType.DMA,
            pltpu.SemaphoreType.DMA,
            pltpu.SemaphoreType.DMA,
            pltpu.SemaphoreType.DMA,
        ],
        compiler_params=pltpu.CompilerParams(
            needs_layout_passes=False, use_tc_tiling_on_sc=False),
    )
    def k(tok_hbm, lutr_hbm, out_hbm, idx_v, idx2_v, rows_0, rows_1,
          blk_a, blk_b, gsem_0, gsem_1, osem_a, osem_b):
        wid = lax.axis_index("s") * nc + lax.axis_index("c")
        pltpu.sync_copy(tok_hbm.at[:, pl.ds(wid * bw, bw)], idx_v)

        lanes = jax.lax.iota(jnp.int32, 16)

        def shift_body(i, carry):
            for g in range(bw // _LANES):
                sl = (i, pl.ds(g * _LANES, _LANES))
                idx2_v[sl] = lax.shift_left(idx_v[sl], 1)
            return carry

        lax.fori_loop(0, l, shift_body, 0)

        def fire_gather(li, rows, gsem):
            return pltpu.async_copy(lutr_hbm.at[idx2_v.at[li]], rows, gsem)

        def wait_gather(rows, gsem):
            pltpu.make_async_copy(lutr_hbm.at[pl.ds(0, bw)], rows, gsem).wait()

        def wait_scatter(blk, osem):
            for dt in range(d // 8):
                pltpu.make_async_copy(
                    out_hbm.at[0, 0, 0], blk.at[pl.ds(dt * 8, 8), :], osem
                ).wait()

        n_grp = bw // _LANES

        def extract(li, rows, blk):
            rowvs = [lanes + (g * _LANES) for g in range(n_grp)]

            def obody(o, carry):
                rvs = carry
                rot = jnp.bitwise_and(lanes + o, 15)
                for c in range(d // _LANES):
                    ddv = rot + (c * _LANES)
                    for g in range(n_grp):
                        val = plsc.load_gather(rows, [rvs[g], ddv])
                        plsc.store_scatter(blk, [ddv, rvs[g]], val)
                return rvs

            lax.fori_loop(0, _LANES, obody, tuple(rowvs))

        def fire_scatter(li, blk, osem):
            for dt in range(d // 8):
                pltpu.async_copy(
                    blk.at[pl.ds(dt * 8, 8), :], out_hbm.at[li, dt, wid], osem
                )

        fire_gather(0, rows_0, gsem_0)
        fire_gather(1, rows_1, gsem_1)

        def pair_body(t, carry):
            l0 = t * 2
            wait_gather(rows_0, gsem_0)

            @pl.when(t > 0)
            def _():
                wait_scatter(blk_a, osem_a)

            extract(l0, rows_0, blk_a)
            fire_scatter(l0, blk_a, osem_a)

            @pl.when(t < n_pairs - 1)
            def _():
                fire_gather(l0 + 2, rows_0, gsem_0)

            wait_gather(rows_1, gsem_1)

            @pl.when(t > 0)
            def _():
                wait_scatter(blk_b, osem_b)

            extract(l0 + 1, rows_1, blk_b)
            fire_scatter(l0 + 1, blk_b, osem_b)

            @pl.when(t < n_pairs - 1)
            def _():
                fire_gather(l0 + 3, rows_1, gsem_1)

            return carry

        lax.fori_loop(0, n_pairs, pair_body, 0)
        wait_scatter(blk_a, osem_a)
        wait_scatter(blk_b, osem_b)

    return k


def kernel(input_tokens, lut):
    b, l = input_tokens.shape
    vocab, d = lut.shape
    tok_t = input_tokens.astype(jnp.int32).T
    lutp = _build_tc_relayout(vocab, d)(lut.T).reshape(2 * vocab, d)
    out5 = _build_sc_lookup(b, l, vocab, d)(tok_t, lutp)
    return out5.transpose(2, 4, 0, 1, 3).reshape(b, l, d)


# speedup vs baseline: 1.4818x; 1.2755x over previous
import functools
import math

import jax
import jax.numpy as jnp
from jax import lax
from jax.experimental import pallas as pl
from jax.experimental.pallas import tpu as pltpu
from jax.experimental.pallas import tpu_sc as plsc

_LANES = 16
_BV = 4096


@functools.lru_cache(maxsize=None)
def _build_tc_relayout(vocab: int, d: int):
    scale = math.sqrt(d)

    def body(in_ref, out_ref):
        out_ref[:, 0:d] = in_ref[...].T * scale

    return pl.pallas_call(
        body,
        grid=(pl.cdiv(vocab, _BV),),
        in_specs=[pl.BlockSpec((d, _BV), lambda i: (0, i))],
        out_specs=pl.BlockSpec((_BV, 2 * d), lambda i: (i, 0)),
        out_shape=jax.ShapeDtypeStruct((vocab, 2 * d), jnp.float32),
    )


@functools.lru_cache(maxsize=None)
def _build_sc_lookup(b: int, l: int, vocab: int, d: int):
    info = plsc.get_sparse_core_info()
    nc, ns = info.num_cores, info.num_subcores
    nw = nc * ns
    bw = b // nw
    assert bw * nw == b and bw == 128
    assert d == 64 and l % 2 == 0
    scale = math.sqrt(d)
    n_pairs = l // 2

    mesh = plsc.VectorSubcoreMesh(core_axis_name="c", subcore_axis_name="s")

    @functools.partial(
        pl.kernel,
        mesh=mesh,
        out_type=jax.ShapeDtypeStruct((l, d // 8, nw, 8, 128), jnp.float32),
        scratch_types=[
            pltpu.VMEM((l, bw), jnp.int32),
            pltpu.VMEM((l, bw), jnp.int32),
            pltpu.VMEM((bw, d), jnp.float32),
            pltpu.VMEM((bw, d), jnp.float32),
            pltpu.VMEM((d, bw), jnp.float32),
            pltpu.VMEM((d, bw), jnp.float32),
            pltpu.SemaphoreType.DMA,
            pltpu.SemaphoreType.DMA,
            pltpu.SemaphoreType.DMA,
            pltpu.SemaphoreType.DMA,
        ],
        compiler_params=pltpu.CompilerParams(
            needs_layout_passes=False, use_tc_tiling_on_sc=False),
    )
    def k(tok_hbm, lutr_hbm, out_hbm, idx_v, idx2_v, rows_0, rows_1,
          blk_a, blk_b, gsem_0, gsem_1, osem_a, osem_b):
        wid = lax.axis_index("s") * nc + lax.axis_index("c")
        pltpu.sync_copy(tok_hbm.at[:, pl.ds(wid * bw, bw)], idx_v)

        lanes = jax.lax.iota(jnp.int32, 16)

        def shift_body(i, carry):
            for g in range(bw // _LANES):
                sl = (i, pl.ds(g * _LANES, _LANES))
                idx2_v[sl] = lax.shift_left(idx_v[sl], 1)
            return carry

        lax.fori_loop(0, l, shift_body, 0)

        def fire_gather(li, rows, gsem):
            return pltpu.async_copy(lutr_hbm.at[idx2_v.at[li]], rows, gsem)

        def wait_gather(rows, gsem):
            pltpu.make_async_copy(lutr_hbm.at[pl.ds(0, bw)], rows, gsem).wait()

        def wait_scatter(blk, osem):
            for dt in range(d // 8):
                pltpu.make_async_copy(
                    out_hbm.at[0, 0, 0], blk.at[pl.ds(dt * 8, 8), :], osem
                ).wait()

        n_grp = bw // _LANES

        def extract(li, rows, blk):
            rowvs = [lanes + (g * _LANES) for g in range(n_grp)]

            def obody(o, carry):
                rvs = carry
                rot = jnp.bitwise_and(lanes + o, 15)
                for c in range(d // _LANES):
                    ddv = rot + (c * _LANES)
                    for g in range(n_grp):
                        val = plsc.load_gather(rows, [rvs[g], ddv])
                        plsc.store_scatter(blk, [ddv, rvs[g]], val)
                return rvs

            lax.fori_loop(0, _LANES, obody, tuple(rowvs))

        def fire_scatter(li, blk, osem):
            for dt in range(d // 8):
                pltpu.async_copy(
                    blk.at[pl.ds(dt * 8, 8), :], out_hbm.at[li, dt, wid], osem
                )

        fire_gather(0, rows_0, gsem_0)
        fire_gather(1, rows_1, gsem_1)

        def pair_body(t, carry):
            l0 = t * 2
            wait_gather(rows_0, gsem_0)

            @pl.when(t > 0)
            def _():
                wait_scatter(blk_a, osem_a)

            extract(l0, rows_0, blk_a)
            fire_scatter(l0, blk_a, osem_a)

            @pl.when(t < n_pairs - 1)
            def _():
                fire_gather(l0 + 2, rows_0, gsem_0)

            wait_gather(rows_1, gsem_1)

            @pl.when(t > 0)
            def _():
                wait_scatter(blk_b, osem_b)

            extract(l0 + 1, rows_1, blk_b)
            fire_scatter(l0 + 1, blk_b, osem_b)

            @pl.when(t < n_pairs - 1)
            def _():
                fire_gather(l0 + 3, rows_1, gsem_1)

            return carry

        lax.fori_loop(0, n_pairs, pair_body, 0)
        wait_scatter(blk_a, osem_a)
        wait_scatter(blk_b, osem_b)

    return k


def kernel(input_tokens, lut):
    b, l = input_tokens.shape
    vocab, d = lut.shape
    tok_t = input_tokens.astype(jnp.int32).T
    lutp = _build_tc_relayout(vocab, d)(lut.T).reshape(2 * vocab, d)
    out5 = _build_sc_lookup(b, l, vocab, d)(tok_t, lutp)
    return out5.transpose(2, 4, 0, 1, 3).reshape(b, l, d)


# speedup vs baseline: 1.7619x; 1.1891x over previous
import functools
import math

import jax
import jax.numpy as jnp
from jax import lax
from jax.experimental import pallas as pl
from jax.experimental.pallas import tpu as pltpu
from jax.experimental.pallas import tpu_sc as plsc

_LANES = 16
_BV = 8192


@functools.lru_cache(maxsize=None)
def _build_tc_relayout(vocab: int, d: int):
    scale = math.sqrt(d)

    def body(in_ref, out_ref):
        out_ref[:, 0:d] = in_ref[...].T * scale

    return pl.pallas_call(
        body,
        grid=(pl.cdiv(vocab, _BV),),
        in_specs=[pl.BlockSpec((d, _BV), lambda i: (0, i))],
        out_specs=pl.BlockSpec((_BV, 2 * d), lambda i: (i, 0)),
        out_shape=jax.ShapeDtypeStruct((vocab, 2 * d), jnp.float32),
    )


@functools.lru_cache(maxsize=None)
def _build_sc_lookup(b: int, l: int, vocab: int, d: int):
    info = plsc.get_sparse_core_info()
    nc, ns = info.num_cores, info.num_subcores
    nw = nc * ns
    bw = b // nw
    assert bw * nw == b and bw == 128
    assert d == 64 and l % 2 == 0
    scale = math.sqrt(d)
    n_pairs = l // 2

    mesh = plsc.VectorSubcoreMesh(core_axis_name="c", subcore_axis_name="s")

    @functools.partial(
        pl.kernel,
        mesh=mesh,
        out_type=jax.ShapeDtypeStruct((l, d // 8, nw, 8, 128), jnp.float32),
        scratch_types=[
            pltpu.VMEM((l, bw), jnp.int32),
            pltpu.VMEM((l, bw), jnp.int32),
            pltpu.VMEM((bw, d), jnp.float32),
            pltpu.VMEM((bw, d), jnp.float32),
            pltpu.VMEM((d, bw), jnp.float32),
            pltpu.VMEM((d, bw), jnp.float32),
            pltpu.SemaphoreType.DMA,
            pltpu.SemaphoreType.DMA,
            pltpu.SemaphoreType.DMA,
            pltpu.SemaphoreType.DMA,
        ],
        compiler_params=pltpu.CompilerParams(
            needs_layout_passes=False, use_tc_tiling_on_sc=False),
    )
    def k(tok_hbm, lutr_hbm, out_hbm, idx_v, idx2_v, rows_0, rows_1,
          blk_a, blk_b, gsem_0, gsem_1, osem_a, osem_b):
        wid = lax.axis_index("s") * nc + lax.axis_index("c")
        pltpu.sync_copy(tok_hbm.at[:, pl.ds(wid * bw, bw)], idx_v)

        lanes = jax.lax.iota(jnp.int32, 16)

        def shift_body(i, carry):
            for g in range(bw // _LANES):
                sl = (i, pl.ds(g * _LANES, _LANES))
                idx2_v[sl] = lax.shift_left(idx_v[sl], 1)
            return carry

        lax.fori_loop(0, l, shift_body, 0)

        def fire_gather(li, rows, gsem):
            return pltpu.async_copy(lutr_hbm.at[idx2_v.at[li]], rows, gsem)

        def wait_gather(rows, gsem):
            pltpu.make_async_copy(lutr_hbm.at[pl.ds(0, bw)], rows, gsem).wait()

        def wait_scatter(blk, osem):
            for dt in range(d // 8):
                pltpu.make_async_copy(
                    out_hbm.at[0, 0, 0], blk.at[pl.ds(dt * 8, 8), :], osem
                ).wait()

        n_grp = bw // _LANES

        def extract(li, rows, blk):
            rowvs = [lanes + (g * _LANES) for g in range(n_grp)]

            def obody(o, carry):
                rvs = carry
                rot = jnp.bitwise_and(lanes + o, 15)
                for c in range(d // _LANES):
                    ddv = rot + (c * _LANES)
                    for g in range(n_grp):
                        val = plsc.load_gather(rows, [rvs[g], ddv])
                        plsc.store_scatter(blk, [ddv, rvs[g]], val)
                return rvs

            lax.fori_loop(0, _LANES, obody, tuple(rowvs))

        def fire_scatter(li, blk, osem):
            for dt in range(d // 8):
                pltpu.async_copy(
                    blk.at[pl.ds(dt * 8, 8), :], out_hbm.at[li, dt, wid], osem
                )

        fire_gather(0, rows_0, gsem_0)
        fire_gather(1, rows_1, gsem_1)

        def pair_body(t, carry):
            l0 = t * 2
            wait_gather(rows_0, gsem_0)

            @pl.when(t > 0)
            def _():
                wait_scatter(blk_a, osem_a)

            extract(l0, rows_0, blk_a)
            fire_scatter(l0, blk_a, osem_a)

            @pl.when(t < n_pairs - 1)
            def _():
                fire_gather(l0 + 2, rows_0, gsem_0)

            wait_gather(rows_1, gsem_1)

            @pl.when(t > 0)
            def _():
                wait_scatter(blk_b, osem_b)

            extract(l0 + 1, rows_1, blk_b)
            fire_scatter(l0 + 1, blk_b, osem_b)

            @pl.when(t < n_pairs - 1)
            def _():
                fire_gather(l0 + 3, rows_1, gsem_1)

            return carry

        lax.fori_loop(0, n_pairs, pair_body, 0)
        wait_scatter(blk_a, osem_a)
        wait_scatter(blk_b, osem_b)

    return k


def kernel(input_tokens, lut):
    b, l = input_tokens.shape
    vocab, d = lut.shape
    tok_t = input_tokens.astype(jnp.int32).T
    lutp = _build_tc_relayout(vocab, d)(lut.T).reshape(2 * vocab, d)
    out5 = _build_sc_lookup(b, l, vocab, d)(tok_t, lutp)
    return out5.transpose(2, 4, 0, 1, 3).reshape(b, l, d)


# speedup vs baseline: 1.8526x; 1.0514x over previous
import functools
import math

import jax
import jax.numpy as jnp
from jax import lax
from jax.experimental import pallas as pl
from jax.experimental.pallas import tpu as pltpu
from jax.experimental.pallas import tpu_sc as plsc

_LANES = 16
_BV = 16384


@functools.lru_cache(maxsize=None)
def _build_tc_relayout(vocab: int, d: int):
    scale = math.sqrt(d)

    def body(in_ref, out_ref):
        out_ref[:, 0:d] = in_ref[...].T * scale

    return pl.pallas_call(
        body,
        grid=(pl.cdiv(vocab, _BV),),
        in_specs=[pl.BlockSpec((d, _BV), lambda i: (0, i))],
        out_specs=pl.BlockSpec((_BV, 2 * d), lambda i: (i, 0)),
        out_shape=jax.ShapeDtypeStruct((vocab, 2 * d), jnp.float32),
    )


@functools.lru_cache(maxsize=None)
def _build_sc_lookup(b: int, l: int, vocab: int, d: int):
    info = plsc.get_sparse_core_info()
    nc, ns = info.num_cores, info.num_subcores
    nw = nc * ns
    bw = b // nw
    assert bw * nw == b and bw == 128
    assert d == 64 and l % 2 == 0
    scale = math.sqrt(d)
    n_pairs = l // 2

    mesh = plsc.VectorSubcoreMesh(core_axis_name="c", subcore_axis_name="s")

    @functools.partial(
        pl.kernel,
        mesh=mesh,
        out_type=jax.ShapeDtypeStruct((l, d // 8, nw, 8, 128), jnp.float32),
        scratch_types=[
            pltpu.VMEM((l, bw), jnp.int32),
            pltpu.VMEM((l, bw), jnp.int32),
            pltpu.VMEM((bw, d), jnp.float32),
            pltpu.VMEM((bw, d), jnp.float32),
            pltpu.VMEM((d, bw), jnp.float32),
            pltpu.VMEM((d, bw), jnp.float32),
            pltpu.SemaphoreType.DMA,
            pltpu.SemaphoreType.DMA,
            pltpu.SemaphoreType.DMA,
            pltpu.SemaphoreType.DMA,
        ],
        compiler_params=pltpu.CompilerParams(
            needs_layout_passes=False, use_tc_tiling_on_sc=False),
    )
    def k(tok_hbm, lutr_hbm, out_hbm, idx_v, idx2_v, rows_0, rows_1,
          blk_a, blk_b, gsem_0, gsem_1, osem_a, osem_b):
        wid = lax.axis_index("s") * nc + lax.axis_index("c")
        pltpu.sync_copy(tok_hbm.at[:, pl.ds(wid * bw, bw)], idx_v)

        lanes = jax.lax.iota(jnp.int32, 16)

        def shift_body(i, carry):
            for g in range(bw // _LANES):
                sl = (i, pl.ds(g * _LANES, _LANES))
                idx2_v[sl] = lax.shift_left(idx_v[sl], 1)
            return carry

        lax.fori_loop(0, l, shift_body, 0)

        def fire_gather(li, rows, gsem):
            return pltpu.async_copy(lutr_hbm.at[idx2_v.at[li]], rows, gsem)

        def wait_gather(rows, gsem):
            pltpu.make_async_copy(lutr_hbm.at[pl.ds(0, bw)], rows, gsem).wait()

        def wait_scatter(blk, osem):
            for dt in range(d // 8):
                pltpu.make_async_copy(
                    out_hbm.at[0, 0, 0], blk.at[pl.ds(dt * 8, 8), :], osem
                ).wait()

        n_grp = bw // _LANES

        def extract(li, rows, blk):
            rowvs = [lanes + (g * _LANES) for g in range(n_grp)]

            def obody(o, carry):
                rvs = carry
                rot = jnp.bitwise_and(lanes + o, 15)
                for c in range(d // _LANES):
                    ddv = rot + (c * _LANES)
                    for g in range(n_grp):
                        val = plsc.load_gather(rows, [rvs[g], ddv])
                        plsc.store_scatter(blk, [ddv, rvs[g]], val)
                return rvs

            lax.fori_loop(0, _LANES, obody, tuple(rowvs))

        def fire_scatter(li, blk, osem):
            for dt in range(d // 8):
                pltpu.async_copy(
                    blk.at[pl.ds(dt * 8, 8), :], out_hbm.at[li, dt, wid], osem
                )

        fire_gather(0, rows_0, gsem_0)
        fire_gather(1, rows_1, gsem_1)

        def pair_body(t, carry):
            l0 = t * 2
            wait_gather(rows_0, gsem_0)

            @pl.when(t > 0)
            def _():
                wait_scatter(blk_a, osem_a)

            extract(l0, rows_0, blk_a)
            fire_scatter(l0, blk_a, osem_a)

            @pl.when(t < n_pairs - 1)
            def _():
                fire_gather(l0 + 2, rows_0, gsem_0)

            wait_gather(rows_1, gsem_1)

            @pl.when(t > 0)
            def _():
                wait_scatter(blk_b, osem_b)

            extract(l0 + 1, rows_1, blk_b)
            fire_scatter(l0 + 1, blk_b, osem_b)

            @pl.when(t < n_pairs - 1)
            def _():
                fire_gather(l0 + 3, rows_1, gsem_1)

            return carry

        lax.fori_loop(0, n_pairs, pair_body, 0)
        wait_scatter(blk_a, osem_a)
        wait_scatter(blk_b, osem_b)

    return k


def kernel(input_tokens, lut):
    b, l = input_tokens.shape
    vocab, d = lut.shape
    tok_t = input_tokens.astype(jnp.int32).T
    lutp = _build_tc_relayout(vocab, d)(lut.T).reshape(2 * vocab, d)
    out5 = _build_sc_lookup(b, l, vocab, d)(tok_t, lutp)
    return out5.transpose(2, 4, 0, 1, 3).reshape(b, l, d)


# speedup vs baseline: 1.8846x; 1.0173x over previous
import functools
import math

import jax
import jax.numpy as jnp
from jax import lax
from jax.experimental import pallas as pl
from jax.experimental.pallas import tpu as pltpu
from jax.experimental.pallas import tpu_sc as plsc

_LANES = 16
_BV = 32768


@functools.lru_cache(maxsize=None)
def _build_tc_relayout(vocab: int, d: int):
    scale = math.sqrt(d)

    def body(in_ref, out_ref):
        out_ref[:, 0:d] = in_ref[...].T * scale

    return pl.pallas_call(
        body,
        grid=(pl.cdiv(vocab, _BV),),
        in_specs=[pl.BlockSpec((d, _BV), lambda i: (0, i))],
        out_specs=pl.BlockSpec((_BV, 2 * d), lambda i: (i, 0)),
        out_shape=jax.ShapeDtypeStruct((vocab, 2 * d), jnp.float32),
    )


@functools.lru_cache(maxsize=None)
def _build_sc_lookup(b: int, l: int, vocab: int, d: int):
    info = plsc.get_sparse_core_info()
    nc, ns = info.num_cores, info.num_subcores
    nw = nc * ns
    bw = b // nw
    assert bw * nw == b and bw == 128
    assert d == 64 and l % 2 == 0
    scale = math.sqrt(d)
    n_pairs = l // 2

    mesh = plsc.VectorSubcoreMesh(core_axis_name="c", subcore_axis_name="s")

    @functools.partial(
        pl.kernel,
        mesh=mesh,
        out_type=jax.ShapeDtypeStruct((l, d // 8, nw, 8, 128), jnp.float32),
        scratch_types=[
            pltpu.VMEM((l, bw), jnp.int32),
            pltpu.VMEM((l, bw), jnp.int32),
            pltpu.VMEM((bw, d), jnp.float32),
            pltpu.VMEM((bw, d), jnp.float32),
            pltpu.VMEM((d, bw), jnp.float32),
            pltpu.VMEM((d, bw), jnp.float32),
            pltpu.SemaphoreType.DMA,
            pltpu.SemaphoreType.DMA,
            pltpu.SemaphoreType.DMA,
            pltpu.SemaphoreType.DMA,
        ],
        compiler_params=pltpu.CompilerParams(
            needs_layout_passes=False, use_tc_tiling_on_sc=False),
    )
    def k(tok_hbm, lutr_hbm, out_hbm, idx_v, idx2_v, rows_0, rows_1,
          blk_a, blk_b, gsem_0, gsem_1, osem_a, osem_b):
        wid = lax.axis_index("s") * nc + lax.axis_index("c")
        pltpu.sync_copy(tok_hbm.at[:, pl.ds(wid * bw, bw)], idx_v)

        lanes = jax.lax.iota(jnp.int32, 16)

        def shift_body(i, carry):
            for g in range(bw // _LANES):
                sl = (i, pl.ds(g * _LANES, _LANES))
                idx2_v[sl] = lax.shift_left(idx_v[sl], 1)
            return carry

        lax.fori_loop(0, l, shift_body, 0)

        def fire_gather(li, rows, gsem):
            return pltpu.async_copy(lutr_hbm.at[idx2_v.at[li]], rows, gsem)

        def wait_gather(rows, gsem):
            pltpu.make_async_copy(lutr_hbm.at[pl.ds(0, bw)], rows, gsem).wait()

        def wait_scatter(blk, osem):
            for dt in range(d // 8):
                pltpu.make_async_copy(
                    out_hbm.at[0, 0, 0], blk.at[pl.ds(dt * 8, 8), :], osem
                ).wait()

        n_grp = bw // _LANES

        def extract(li, rows, blk):
            rowvs = [lanes + (g * _LANES) for g in range(n_grp)]

            def obody(o, carry):
                rvs = carry
                rot = jnp.bitwise_and(lanes + o, 15)
                for c in range(d // _LANES):
                    ddv = rot + (c * _LANES)
                    for g in range(n_grp):
                        val = plsc.load_gather(rows, [rvs[g], ddv])
                        plsc.store_scatter(blk, [ddv, rvs[g]], val)
                return rvs

            lax.fori_loop(0, _LANES, obody, tuple(rowvs))

        def fire_scatter(li, blk, osem):
            for dt in range(d // 8):
                pltpu.async_copy(
                    blk.at[pl.ds(dt * 8, 8), :], out_hbm.at[li, dt, wid], osem
                )

        fire_gather(0, rows_0, gsem_0)
        fire_gather(1, rows_1, gsem_1)

        def pair_body(t, carry):
            l0 = t * 2
            wait_gather(rows_0, gsem_0)

            @pl.when(t > 0)
            def _():
                wait_scatter(blk_a, osem_a)

            extract(l0, rows_0, blk_a)
            fire_scatter(l0, blk_a, osem_a)

            @pl.when(t < n_pairs - 1)
            def _():
                fire_gather(l0 + 2, rows_0, gsem_0)

            wait_gather(rows_1, gsem_1)

            @pl.when(t > 0)
            def _():
                wait_scatter(blk_b, osem_b)

            extract(l0 + 1, rows_1, blk_b)
            fire_scatter(l0 + 1, blk_b, osem_b)

            @pl.when(t < n_pairs - 1)
            def _():
                fire_gather(l0 + 3, rows_1, gsem_1)

            return carry

        lax.fori_loop(0, n_pairs, pair_body, 0)
        wait_scatter(blk_a, osem_a)
        wait_scatter(blk_b, osem_b)

    return k


def kernel(input_tokens, lut):
    b, l = input_tokens.shape
    vocab, d = lut.shape
    tok_t = input_tokens.astype(jnp.int32).T
    lutp = _build_tc_relayout(vocab, d)(lut.T).reshape(2 * vocab, d)
    out5 = _build_sc_lookup(b, l, vocab, d)(tok_t, lutp)
    return out5.transpose(2, 4, 0, 1, 3).reshape(b, l, d)
